# Initial kernel scaffold; baseline (speedup 1.0000x reference)
import jax
import jax.numpy as jnp
from jax.experimental import pallas as pl


def kernel(x_paper, x_author, edge_index_writes, edge_index_rev_writes, W_paper_0, W_paper_1, W_author_0, W_author_1, W_out, b_out):
    raise NotImplementedError("write your pallas kernel here")


# speedup vs baseline: 6.5125x; 6.5125x over previous
import functools

import jax
import jax.numpy as jnp
from jax import lax
from jax.experimental import pallas as pl
from jax.experimental.pallas import tpu as pltpu
from jax.experimental.pallas import tpu_sc as plsc

N = 10000
DH = 128
E = 320000
NC = 2
NS = 16
NW = NC * NS
EPW = E // NW
K = 80
NCHUNK = EPW // K
RPS = 624
RREM = N - NS * RPS

_mesh = plsc.VectorSubcoreMesh(core_axis_name="c", subcore_axis_name="s")


@functools.partial(
    pl.kernel,
    out_type=(
        jax.ShapeDtypeStruct((N, DH), jnp.float32),
        jax.ShapeDtypeStruct((N, DH), jnp.float32),
    ),
    mesh=_mesh,
    scratch_types=[
        pltpu.VMEM_SHARED((N, DH), jnp.float32),
        pltpu.VMEM((NCHUNK, K), jnp.int32),
        pltpu.VMEM((NCHUNK, K), jnp.int32),
        pltpu.VMEM((K, DH), jnp.float32),
        pltpu.SemaphoreType.DMA,
    ],
)
def _spmm_kernel(h_src, h_init, src2d, dst2d, out0, out1,
                 acc, idxs, idxd, rows, sem):
    cid = lax.axis_index("c")
    sid = lax.axis_index("s")
    wid = sid * NC + cid

    rbase = sid * RPS
    pltpu.sync_copy(h_init.at[pl.ds(rbase, RPS)], acc.at[pl.ds(rbase, RPS)])

    @pl.when(sid == NS - 1)
    def _():
        pltpu.sync_copy(h_init.at[pl.ds(NS * RPS, RREM)],
                        acc.at[pl.ds(NS * RPS, RREM)])

    pltpu.sync_copy(src2d.at[wid], idxs)
    pltpu.sync_copy(dst2d.at[wid], idxd)
    plsc.subcore_barrier()

    def step(i, _):
        pltpu.async_copy(h_src.at[idxs.at[i]], rows, sem).wait()
        pltpu.sync_copy(rows, acc.at[idxd.at[i]], add=True)
        return ()

    lax.fori_loop(0, NCHUNK, step, (), unroll=False)
    plsc.subcore_barrier()

    @pl.when(cid == 0)
    def _():
        pltpu.sync_copy(acc.at[pl.ds(rbase, RPS)], out0.at[pl.ds(rbase, RPS)])

        @pl.when(sid == NS - 1)
        def _():
            pltpu.sync_copy(acc.at[pl.ds(NS * RPS, RREM)],
                            out0.at[pl.ds(NS * RPS, RREM)])

    @pl.when(cid == 1)
    def _():
        pltpu.sync_copy(acc.at[pl.ds(rbase, RPS)], out1.at[pl.ds(rbase, RPS)])

        @pl.when(sid == NS - 1)
        def _():
            pltpu.sync_copy(acc.at[pl.ds(NS * RPS, RREM)],
                            out1.at[pl.ds(NS * RPS, RREM)])


def _spmm(h_src, h_init, src2d, dst2d):
    return _spmm_kernel(h_src, h_init, src2d, dst2d)


_BM = 2000


def _mm_body(x_ref, w_ref, o_ref):
    o_ref[...] = jnp.maximum(
        jnp.dot(x_ref[...], w_ref[...], preferred_element_type=jnp.float32),
        0.0)


def _mm_relu(x, w):
    m, d = x.shape
    dout = w.shape[1]
    return pl.pallas_call(
        _mm_body,
        grid=(m // _BM,),
        in_specs=[
            pl.BlockSpec((_BM, d), lambda i: (i, 0)),
            pl.BlockSpec((d, dout), lambda i: (0, 0)),
        ],
        out_specs=pl.BlockSpec((_BM, dout), lambda i: (i, 0)),
        out_shape=jax.ShapeDtypeStruct((m, dout), jnp.float32),
    )(x, w)


def _mm3_relu_body(x0_ref, x1_ref, x2_ref, w_ref, o_ref):
    xs = x0_ref[...] + x1_ref[...] - x2_ref[...]
    o_ref[...] = jnp.maximum(
        jnp.dot(xs, w_ref[...] * 0.5, preferred_element_type=jnp.float32),
        0.0)


def _mm3_bias_body(x0_ref, x1_ref, x2_ref, w_ref, b_ref, o_ref):
    xs = x0_ref[...] + x1_ref[...] - x2_ref[...]
    o_ref[...] = jnp.dot(
        xs, w_ref[...] * 0.5, preferred_element_type=jnp.float32) + b_ref[...]


def _mm3(x0, x1, x2, w, bias=None):
    m, d = x0.shape
    dout = w.shape[1]
    xspec = pl.BlockSpec((_BM, d), lambda i: (i, 0))
    in_specs = [xspec, xspec, xspec, pl.BlockSpec((d, dout), lambda i: (0, 0))]
    operands = [x0, x1, x2, w]
    if bias is None:
        body = _mm3_relu_body
    else:
        body = _mm3_bias_body
        in_specs.append(pl.BlockSpec((1, dout), lambda i: (0, 0)))
        operands.append(bias.reshape(1, dout))
    return pl.pallas_call(
        body,
        grid=(m // _BM,),
        in_specs=in_specs,
        out_specs=pl.BlockSpec((_BM, dout), lambda i: (i, 0)),
        out_shape=jax.ShapeDtypeStruct((m, dout), jnp.float32),
    )(*operands)


def kernel(x_paper, x_author, edge_index_writes, edge_index_rev_writes,
           W_paper_0, W_paper_1, W_author_0, W_author_1, W_out, b_out):
    src_w = edge_index_writes[0].reshape(NW, NCHUNK, K)
    dst_w = edge_index_writes[1].reshape(NW, NCHUNK, K)
    src_r = edge_index_rev_writes[0].reshape(NW, NCHUNK, K)
    dst_r = edge_index_rev_writes[1].reshape(NW, NCHUNK, K)

    h_p0 = _mm_relu(x_paper, W_paper_0)
    h_a0 = _mm_relu(x_author, W_author_0)
    p0, p1 = _spmm(h_a0, h_p0, src_w, dst_w)
    a0, a1 = _spmm(h_p0, h_a0, src_r, dst_r)
    h_p1 = _mm3(p0, p1, h_p0, W_paper_1)
    h_a1 = _mm3(a0, a1, h_a0, W_author_1)
    q0, q1 = _spmm(h_a1, h_p1, src_w, dst_w)
    return _mm3(q0, q1, h_p1, W_out, bias=b_out)


# speedup vs baseline: 8.0042x; 1.2290x over previous
import functools

import jax
import jax.numpy as jnp
from jax import lax
from jax.experimental import pallas as pl
from jax.experimental.pallas import tpu as pltpu
from jax.experimental.pallas import tpu_sc as plsc

N = 10000
DH = 128
KH = DH // 2
E = 320000
NC = 2
NS = 16
EPT = E // NS
K = 80
NCHT = EPT // K
RPS = 624
RREM = N - NS * RPS

_mesh = plsc.VectorSubcoreMesh(core_axis_name="c", subcore_axis_name="s")


@functools.partial(
    pl.kernel,
    out_type=(
        jax.ShapeDtypeStruct((N, KH), jnp.float32),
        jax.ShapeDtypeStruct((N, KH), jnp.float32),
    ),
    mesh=_mesh,
    scratch_types=[
        pltpu.VMEM_SHARED((N, KH), jnp.float32),
        pltpu.VMEM((NCHT, K), jnp.int32),
        pltpu.VMEM((NCHT, K), jnp.int32),
        pltpu.VMEM((K, KH), jnp.float32),
        pltpu.VMEM((K, KH), jnp.float32),
        pltpu.SemaphoreType.DMA,
        pltpu.SemaphoreType.DMA,
    ],
    compiler_params=pltpu.CompilerParams(use_tc_tiling_on_sc=False),
)
def _spmm_kernel(hs0, hs1, hi0, hi1, src3d, dst3d, out0, out1,
                 acc, idxs, idxd, rows0, rows1, sem0, sem1):
    cid = lax.axis_index("c")
    sid = lax.axis_index("s")
    rbase = sid * RPS

    pltpu.sync_copy(src3d.at[sid], idxs)
    pltpu.sync_copy(dst3d.at[sid], idxd)

    def run(h_src, h_init, out):
        pltpu.sync_copy(h_init.at[pl.ds(rbase, RPS)],
                        acc.at[pl.ds(rbase, RPS)])

        @pl.when(sid == NS - 1)
        def _():
            pltpu.sync_copy(h_init.at[pl.ds(NS * RPS, RREM)],
                            acc.at[pl.ds(NS * RPS, RREM)])

        plsc.subcore_barrier()

        pltpu.async_copy(h_src.at[idxs.at[0]], rows0, sem0)

        def step(j, _):
            c0 = 2 * j
            c1 = 2 * j + 1
            pltpu.async_copy(h_src.at[idxs.at[c1]], rows1, sem1)
            pltpu.make_async_copy(h_src.at[idxs.at[c0]], rows0, sem0).wait()
            pltpu.sync_copy(rows0, acc.at[idxd.at[c0]], add=True)

            @pl.when(c1 + 1 < NCHT)
            def _():
                pltpu.async_copy(h_src.at[idxs.at[c1 + 1]], rows0, sem0)

            pltpu.make_async_copy(h_src.at[idxs.at[c1]], rows1, sem1).wait()
            pltpu.sync_copy(rows1, acc.at[idxd.at[c1]], add=True)
            return ()

        lax.fori_loop(0, NCHT // 2, step, (), unroll=False)
        plsc.subcore_barrier()

        pltpu.sync_copy(acc.at[pl.ds(rbase, RPS)], out.at[pl.ds(rbase, RPS)])

        @pl.when(sid == NS - 1)
        def _():
            pltpu.sync_copy(acc.at[pl.ds(NS * RPS, RREM)],
                            out.at[pl.ds(NS * RPS, RREM)])

    @pl.when(cid == 0)
    def _():
        run(hs0, hi0, out0)

    @pl.when(cid == 1)
    def _():
        run(hs1, hi1, out1)


_BM = 2000


def _mm_in_body(x_ref, w_ref, o0_ref, o1_ref):
    h = jnp.maximum(
        jnp.dot(x_ref[...], w_ref[...], preferred_element_type=jnp.float32),
        0.0)
    o0_ref[...] = h[:, :KH]
    o1_ref[...] = h[:, KH:]


def _mm_in(x, w):
    m = x.shape[0]
    half = jax.ShapeDtypeStruct((m, KH), jnp.float32)
    ospec = pl.BlockSpec((_BM, KH), lambda i: (i, 0))
    return pl.pallas_call(
        _mm_in_body,
        grid=(m // _BM,),
        in_specs=[
            pl.BlockSpec((_BM, DH), lambda i: (i, 0)),
            pl.BlockSpec((DH, DH), lambda i: (0, 0)),
        ],
        out_specs=(ospec, ospec),
        out_shape=(half, half),
    )(x, w)


def _mm_mid_body(x0_ref, x1_ref, w_ref, o0_ref, o1_ref):
    x = jnp.concatenate([x0_ref[...], x1_ref[...]], axis=1)
    h = jnp.maximum(
        jnp.dot(x, w_ref[...] * 0.5, preferred_element_type=jnp.float32),
        0.0)
    o0_ref[...] = h[:, :KH]
    o1_ref[...] = h[:, KH:]


def _mm_mid(x0, x1, w):
    m = x0.shape[0]
    half = jax.ShapeDtypeStruct((m, KH), jnp.float32)
    xspec = pl.BlockSpec((_BM, KH), lambda i: (i, 0))
    return pl.pallas_call(
        _mm_mid_body,
        grid=(m // _BM,),
        in_specs=[xspec, xspec, pl.BlockSpec((DH, DH), lambda i: (0, 0))],
        out_specs=(xspec, xspec),
        out_shape=(half, half),
    )(x0, x1, w)


def _mm_head_body(x0_ref, x1_ref, w_ref, b_ref, o_ref):
    x = jnp.concatenate([x0_ref[...], x1_ref[...]], axis=1)
    o_ref[...] = jnp.dot(
        x, w_ref[...] * 0.5, preferred_element_type=jnp.float32) + b_ref[...]


def _mm_head(x0, x1, w, b):
    m = x0.shape[0]
    c = w.shape[1]
    xspec = pl.BlockSpec((_BM, KH), lambda i: (i, 0))
    return pl.pallas_call(
        _mm_head_body,
        grid=(m // _BM,),
        in_specs=[
            xspec, xspec,
            pl.BlockSpec((DH, c), lambda i: (0, 0)),
            pl.BlockSpec((1, c), lambda i: (0, 0)),
        ],
        out_specs=pl.BlockSpec((_BM, c), lambda i: (i, 0)),
        out_shape=jax.ShapeDtypeStruct((m, c), jnp.float32),
    )(x0, x1, w, b.reshape(1, c))


def kernel(x_paper, x_author, edge_index_writes, edge_index_rev_writes,
           W_paper_0, W_paper_1, W_author_0, W_author_1, W_out, b_out):
    src_w = edge_index_writes[0].reshape(NS, NCHT, K)
    dst_w = edge_index_writes[1].reshape(NS, NCHT, K)
    src_r = edge_index_rev_writes[0].reshape(NS, NCHT, K)
    dst_r = edge_index_rev_writes[1].reshape(NS, NCHT, K)

    hp0_0, hp0_1 = _mm_in(x_paper, W_paper_0)
    ha0_0, ha0_1 = _mm_in(x_author, W_author_0)
    pp_0, pp_1 = _spmm_kernel(ha0_0, ha0_1, hp0_0, hp0_1, src_w, dst_w)
    pa_0, pa_1 = _spmm_kernel(hp0_0, hp0_1, ha0_0, ha0_1, src_r, dst_r)
    hp1_0, hp1_1 = _mm_mid(pp_0, pp_1, W_paper_1)
    ha1_0, ha1_1 = _mm_mid(pa_0, pa_1, W_author_1)
    qq_0, qq_1 = _spmm_kernel(ha1_0, ha1_1, hp1_0, hp1_1, src_w, dst_w)
    return _mm_head(qq_0, qq_1, W_out, b_out)


# speedup vs baseline: 9.3468x; 1.1677x over previous
import functools

import jax
import jax.numpy as jnp
from jax import lax
from jax.experimental import pallas as pl
from jax.experimental.pallas import tpu as pltpu
from jax.experimental.pallas import tpu_sc as plsc

N = 10000
DH = 128
KH = DH // 2
E = 320000
NC = 2
NS = 16
EPT = E // NS
K = 80
NCHT = EPT // K
RPS = 624
RREM = N - NS * RPS

_mesh = plsc.VectorSubcoreMesh(core_axis_name="c", subcore_axis_name="s")


@functools.partial(
    pl.kernel,
    out_type=(
        jax.ShapeDtypeStruct((N, KH), jnp.float32),
        jax.ShapeDtypeStruct((N, KH), jnp.float32),
    ),
    mesh=_mesh,
    scratch_types=[
        pltpu.VMEM_SHARED((N, KH), jnp.float32),
        pltpu.VMEM((NCHT, K), jnp.int32),
        pltpu.VMEM((NCHT, K), jnp.int32),
        pltpu.VMEM((K, KH), jnp.float32),
        pltpu.VMEM((K, KH), jnp.float32),
        pltpu.VMEM((K, KH), jnp.float32),
        pltpu.SemaphoreType.DMA,
        pltpu.SemaphoreType.DMA,
        pltpu.SemaphoreType.DMA,
        pltpu.SemaphoreType.DMA,
        pltpu.SemaphoreType.DMA,
        pltpu.SemaphoreType.DMA,
    ],
    compiler_params=pltpu.CompilerParams(use_tc_tiling_on_sc=False),
)
def _spmm_kernel(hs0, hs1, hi0, hi1, src3d, dst3d, out0, out1,
                 acc, idxs, idxd, rows0, rows1, rows2,
                 sg0, sg1, sg2, ss0, ss1, ss2):
    cid = lax.axis_index("c")
    sid = lax.axis_index("s")
    rbase = sid * RPS

    pltpu.sync_copy(src3d.at[sid], idxs)
    pltpu.sync_copy(dst3d.at[sid], idxd)

    def run(h_src, h_init, out):
        pltpu.sync_copy(h_init.at[pl.ds(rbase, RPS)],
                        acc.at[pl.ds(rbase, RPS)])

        @pl.when(sid == NS - 1)
        def _():
            pltpu.sync_copy(h_init.at[pl.ds(NS * RPS, RREM)],
                            acc.at[pl.ds(NS * RPS, RREM)])

        plsc.subcore_barrier()

        bufs = (rows0, rows1, rows2)
        gsems = (sg0, sg1, sg2)
        ssems = (ss0, ss1, ss2)

        def substep(i, b):
            buf, b2 = bufs[b], (b + 2) % 3

            @pl.when(i < NCHT)
            def _():
                pltpu.make_async_copy(
                    h_src.at[idxs.at[i]], buf, gsems[b]).wait()
                pltpu.async_copy(buf, acc.at[idxd.at[i]], ssems[b], add=True)

                @pl.when(i + 2 < NCHT)
                def _():
                    @pl.when(i >= 1)
                    def _():
                        pltpu.make_async_copy(
                            bufs[b2], acc.at[idxd.at[i - 1]],
                            ssems[b2]).wait()

                    pltpu.async_copy(
                        h_src.at[idxs.at[i + 2]], bufs[b2], gsems[b2])

        pltpu.async_copy(h_src.at[idxs.at[0]], rows0, sg0)
        pltpu.async_copy(h_src.at[idxs.at[1]], rows1, sg1)

        def step(j, _):
            substep(3 * j, 0)
            substep(3 * j + 1, 1)
            substep(3 * j + 2, 2)
            return ()

        lax.fori_loop(0, (NCHT + 2) // 3, step, (), unroll=False)
        for i in range(NCHT - 3, NCHT):
            b = i % 3
            pltpu.make_async_copy(
                bufs[b], acc.at[idxd.at[i]], ssems[b]).wait()
        plsc.subcore_barrier()

        pltpu.sync_copy(acc.at[pl.ds(rbase, RPS)], out.at[pl.ds(rbase, RPS)])

        @pl.when(sid == NS - 1)
        def _():
            pltpu.sync_copy(acc.at[pl.ds(NS * RPS, RREM)],
                            out.at[pl.ds(NS * RPS, RREM)])

    @pl.when(cid == 0)
    def _():
        run(hs0, hi0, out0)

    @pl.when(cid == 1)
    def _():
        run(hs1, hi1, out1)


_BM = 2000


def _mm_in_body(x_ref, w_ref, o0_ref, o1_ref):
    h = jnp.maximum(
        jnp.dot(x_ref[...], w_ref[...], preferred_element_type=jnp.float32),
        0.0)
    o0_ref[...] = h[:, :KH]
    o1_ref[...] = h[:, KH:]


def _mm_in(x, w):
    m = x.shape[0]
    half = jax.ShapeDtypeStruct((m, KH), jnp.float32)
    ospec = pl.BlockSpec((_BM, KH), lambda i: (i, 0))
    return pl.pallas_call(
        _mm_in_body,
        grid=(m // _BM,),
        in_specs=[
            pl.BlockSpec((_BM, DH), lambda i: (i, 0)),
            pl.BlockSpec((DH, DH), lambda i: (0, 0)),
        ],
        out_specs=(ospec, ospec),
        out_shape=(half, half),
    )(x, w)


def _mm_mid_body(x0_ref, x1_ref, w_ref, o0_ref, o1_ref):
    x = jnp.concatenate([x0_ref[...], x1_ref[...]], axis=1)
    h = jnp.maximum(
        jnp.dot(x, w_ref[...] * 0.5, preferred_element_type=jnp.float32),
        0.0)
    o0_ref[...] = h[:, :KH]
    o1_ref[...] = h[:, KH:]


def _mm_mid(x0, x1, w):
    m = x0.shape[0]
    half = jax.ShapeDtypeStruct((m, KH), jnp.float32)
    xspec = pl.BlockSpec((_BM, KH), lambda i: (i, 0))
    return pl.pallas_call(
        _mm_mid_body,
        grid=(m // _BM,),
        in_specs=[xspec, xspec, pl.BlockSpec((DH, DH), lambda i: (0, 0))],
        out_specs=(xspec, xspec),
        out_shape=(half, half),
    )(x0, x1, w)


def _mm_head_body(x0_ref, x1_ref, w_ref, b_ref, o_ref):
    x = jnp.concatenate([x0_ref[...], x1_ref[...]], axis=1)
    o_ref[...] = jnp.dot(
        x, w_ref[...] * 0.5, preferred_element_type=jnp.float32) + b_ref[...]


def _mm_head(x0, x1, w, b):
    m = x0.shape[0]
    c = w.shape[1]
    xspec = pl.BlockSpec((_BM, KH), lambda i: (i, 0))
    return pl.pallas_call(
        _mm_head_body,
        grid=(m // _BM,),
        in_specs=[
            xspec, xspec,
            pl.BlockSpec((DH, c), lambda i: (0, 0)),
            pl.BlockSpec((1, c), lambda i: (0, 0)),
        ],
        out_specs=pl.BlockSpec((_BM, c), lambda i: (i, 0)),
        out_shape=jax.ShapeDtypeStruct((m, c), jnp.float32),
    )(x0, x1, w, b.reshape(1, c))


def kernel(x_paper, x_author, edge_index_writes, edge_index_rev_writes,
           W_paper_0, W_paper_1, W_author_0, W_author_1, W_out, b_out):
    src_w = edge_index_writes[0].reshape(NS, NCHT, K)
    dst_w = edge_index_writes[1].reshape(NS, NCHT, K)
    src_r = edge_index_rev_writes[0].reshape(NS, NCHT, K)
    dst_r = edge_index_rev_writes[1].reshape(NS, NCHT, K)

    hp0_0, hp0_1 = _mm_in(x_paper, W_paper_0)
    ha0_0, ha0_1 = _mm_in(x_author, W_author_0)
    pp_0, pp_1 = _spmm_kernel(ha0_0, ha0_1, hp0_0, hp0_1, src_w, dst_w)
    pa_0, pa_1 = _spmm_kernel(hp0_0, hp0_1, ha0_0, ha0_1, src_r, dst_r)
    hp1_0, hp1_1 = _mm_mid(pp_0, pp_1, W_paper_1)
    ha1_0, ha1_1 = _mm_mid(pa_0, pa_1, W_author_1)
    qq_0, qq_1 = _spmm_kernel(ha1_0, ha1_1, hp1_0, hp1_1, src_w, dst_w)
    return _mm_head(qq_0, qq_1, W_out, b_out)


# speedup vs baseline: 10.0394x; 1.0741x over previous
import functools

import jax
import jax.numpy as jnp
from jax import lax
from jax.experimental import pallas as pl
from jax.experimental.pallas import tpu as pltpu
from jax.experimental.pallas import tpu_sc as plsc

N = 10000
DH = 128
KH = DH // 2
E = 320000
NC = 2
NS = 16
EPT = E // NS
K = 80
NCHT = EPT // K
RPS = 624
RREM = N - NS * RPS

_mesh = plsc.VectorSubcoreMesh(core_axis_name="c", subcore_axis_name="s")


@functools.partial(
    pl.kernel,
    out_type=(
        jax.ShapeDtypeStruct((N, KH), jnp.float32),
        jax.ShapeDtypeStruct((N, KH), jnp.float32),
    ),
    mesh=_mesh,
    scratch_types=[
        pltpu.VMEM_SHARED((N, KH), jnp.float32),
        pltpu.VMEM((NCHT, K), jnp.int32),
        pltpu.VMEM((NCHT, K), jnp.int32),
    ] + [pltpu.VMEM((K, KH), jnp.float32) for _ in range(6)]
      + [pltpu.SemaphoreType.DMA for _ in range(12)],
    compiler_params=pltpu.CompilerParams(use_tc_tiling_on_sc=False),
)
def _spmm_kernel(hs0, hs1, hi0, hi1, src3d, dst3d, out0, out1,
                 acc, idxs, idxd, *ring):
    cid = lax.axis_index("c")
    sid = lax.axis_index("s")
    rbase = sid * RPS

    pltpu.sync_copy(src3d.at[sid], idxs)
    pltpu.sync_copy(dst3d.at[sid], idxd)

    def run(h_src, h_init, out):
        pltpu.sync_copy(h_init.at[pl.ds(rbase, RPS)],
                        acc.at[pl.ds(rbase, RPS)])

        @pl.when(sid == NS - 1)
        def _():
            pltpu.sync_copy(h_init.at[pl.ds(NS * RPS, RREM)],
                            acc.at[pl.ds(NS * RPS, RREM)])

        plsc.subcore_barrier()

        B, S = 6, 3
        bufs = ring[:B]
        gsems = ring[B:2 * B]
        ssems = ring[2 * B:3 * B]

        def substep(i, b):
            buf, b2 = bufs[b], (b - S) % B

            @pl.when(i < NCHT)
            def _():
                pltpu.make_async_copy(
                    h_src.at[idxs.at[i]], buf, gsems[b]).wait()
                pltpu.async_copy(buf, acc.at[idxd.at[i]], ssems[b], add=True)

                @pl.when(i + B - S < NCHT)
                def _():
                    @pl.when(i >= S)
                    def _():
                        pltpu.make_async_copy(
                            bufs[b2], acc.at[idxd.at[i - S]],
                            ssems[b2]).wait()

                    pltpu.async_copy(
                        h_src.at[idxs.at[i + B - S]], bufs[b2], gsems[b2])

        for i in range(B - S):
            pltpu.async_copy(h_src.at[idxs.at[i]], bufs[i], gsems[i])

        def step(j, _):
            for b in range(B):
                substep(B * j + b, b)
            return ()

        lax.fori_loop(0, (NCHT + B - 1) // B, step, (), unroll=False)
        for i in range(NCHT - B, NCHT):
            b = i % B
            pltpu.make_async_copy(
                bufs[b], acc.at[idxd.at[i]], ssems[b]).wait()
        plsc.subcore_barrier()

        pltpu.sync_copy(acc.at[pl.ds(rbase, RPS)], out.at[pl.ds(rbase, RPS)])

        @pl.when(sid == NS - 1)
        def _():
            pltpu.sync_copy(acc.at[pl.ds(NS * RPS, RREM)],
                            out.at[pl.ds(NS * RPS, RREM)])

    @pl.when(cid == 0)
    def _():
        run(hs0, hi0, out0)

    @pl.when(cid == 1)
    def _():
        run(hs1, hi1, out1)


_BM = 2000


def _mm_in_body(x_ref, w_ref, o0_ref, o1_ref):
    h = jnp.maximum(
        jnp.dot(x_ref[...], w_ref[...], preferred_element_type=jnp.float32),
        0.0)
    o0_ref[...] = h[:, :KH]
    o1_ref[...] = h[:, KH:]


def _mm_in(x, w):
    m = x.shape[0]
    half = jax.ShapeDtypeStruct((m, KH), jnp.float32)
    ospec = pl.BlockSpec((_BM, KH), lambda i: (i, 0))
    return pl.pallas_call(
        _mm_in_body,
        grid=(m // _BM,),
        in_specs=[
            pl.BlockSpec((_BM, DH), lambda i: (i, 0)),
            pl.BlockSpec((DH, DH), lambda i: (0, 0)),
        ],
        out_specs=(ospec, ospec),
        out_shape=(half, half),
    )(x, w)


def _mm_mid_body(x0_ref, x1_ref, w_ref, o0_ref, o1_ref):
    x = jnp.concatenate([x0_ref[...], x1_ref[...]], axis=1)
    h = jnp.maximum(
        jnp.dot(x, w_ref[...] * 0.5, preferred_element_type=jnp.float32),
        0.0)
    o0_ref[...] = h[:, :KH]
    o1_ref[...] = h[:, KH:]


def _mm_mid(x0, x1, w):
    m = x0.shape[0]
    half = jax.ShapeDtypeStruct((m, KH), jnp.float32)
    xspec = pl.BlockSpec((_BM, KH), lambda i: (i, 0))
    return pl.pallas_call(
        _mm_mid_body,
        grid=(m // _BM,),
        in_specs=[xspec, xspec, pl.BlockSpec((DH, DH), lambda i: (0, 0))],
        out_specs=(xspec, xspec),
        out_shape=(half, half),
    )(x0, x1, w)


def _mm_head_body(x0_ref, x1_ref, w_ref, b_ref, o_ref):
    x = jnp.concatenate([x0_ref[...], x1_ref[...]], axis=1)
    o_ref[...] = jnp.dot(
        x, w_ref[...] * 0.5, preferred_element_type=jnp.float32) + b_ref[...]


def _mm_head(x0, x1, w, b):
    m = x0.shape[0]
    c = w.shape[1]
    xspec = pl.BlockSpec((_BM, KH), lambda i: (i, 0))
    return pl.pallas_call(
        _mm_head_body,
        grid=(m // _BM,),
        in_specs=[
            xspec, xspec,
            pl.BlockSpec((DH, c), lambda i: (0, 0)),
            pl.BlockSpec((1, c), lambda i: (0, 0)),
        ],
        out_specs=pl.BlockSpec((_BM, c), lambda i: (i, 0)),
        out_shape=jax.ShapeDtypeStruct((m, c), jnp.float32),
    )(x0, x1, w, b.reshape(1, c))


def kernel(x_paper, x_author, edge_index_writes, edge_index_rev_writes,
           W_paper_0, W_paper_1, W_author_0, W_author_1, W_out, b_out):
    src_w = edge_index_writes[0].reshape(NS, NCHT, K)
    dst_w = edge_index_writes[1].reshape(NS, NCHT, K)
    src_r = edge_index_rev_writes[0].reshape(NS, NCHT, K)
    dst_r = edge_index_rev_writes[1].reshape(NS, NCHT, K)

    hp0_0, hp0_1 = _mm_in(x_paper, W_paper_0)
    ha0_0, ha0_1 = _mm_in(x_author, W_author_0)
    pp_0, pp_1 = _spmm_kernel(ha0_0, ha0_1, hp0_0, hp0_1, src_w, dst_w)
    pa_0, pa_1 = _spmm_kernel(hp0_0, hp0_1, ha0_0, ha0_1, src_r, dst_r)
    hp1_0, hp1_1 = _mm_mid(pp_0, pp_1, W_paper_1)
    ha1_0, ha1_1 = _mm_mid(pa_0, pa_1, W_author_1)
    qq_0, qq_1 = _spmm_kernel(ha1_0, ha1_1, hp1_0, hp1_1, src_w, dst_w)
    return _mm_head(qq_0, qq_1, W_out, b_out)


# speedup vs baseline: 10.7746x; 1.0732x over previous
import functools

import jax
import jax.numpy as jnp
from jax import lax
from jax.experimental import pallas as pl
from jax.experimental.pallas import tpu as pltpu
from jax.experimental.pallas import tpu_sc as plsc

N = 10000
DH = 128
KH = DH // 2
E = 320000
NC = 2
NS = 16
EPT = E // NS
K = 80
NCHT = EPT // K
RPS = 624
RREM = N - NS * RPS

_mesh = plsc.VectorSubcoreMesh(core_axis_name="c", subcore_axis_name="s")


@functools.partial(
    pl.kernel,
    out_type=(
        jax.ShapeDtypeStruct((N, KH), jnp.float32),
        jax.ShapeDtypeStruct((N, KH), jnp.float32),
    ),
    mesh=_mesh,
    scratch_types=[
        pltpu.VMEM_SHARED((N, KH), jnp.float32),
        pltpu.VMEM((NCHT, K), jnp.int32),
        pltpu.VMEM((NCHT, K), jnp.int32),
    ] + [pltpu.VMEM((K, KH), jnp.float32) for _ in range(8)]
      + [pltpu.SemaphoreType.DMA for _ in range(17)],
    compiler_params=pltpu.CompilerParams(use_tc_tiling_on_sc=False),
)
def _spmm_kernel(hs0, hs1, hi0, hi1, src3d, dst3d, out0, out1,
                 acc, idxs, idxd, *ring):
    cid = lax.axis_index("c")
    sid = lax.axis_index("s")
    rbase = sid * RPS

    isem = ring[-1]
    pltpu.async_copy(src3d.at[sid], idxs, isem)
    pltpu.async_copy(dst3d.at[sid], idxd, isem)

    def run(h_src, h_init, out):
        pltpu.sync_copy(h_init.at[pl.ds(rbase, RPS)],
                        acc.at[pl.ds(rbase, RPS)])

        @pl.when(sid == NS - 1)
        def _():
            pltpu.sync_copy(h_init.at[pl.ds(NS * RPS, RREM)],
                            acc.at[pl.ds(NS * RPS, RREM)])

        plsc.subcore_barrier()

        B, S = 8, 4
        bufs = ring[:B]
        gsems = ring[B:2 * B]
        ssems = ring[2 * B:3 * B]
        pltpu.make_async_copy(src3d.at[sid], idxs, isem).wait()
        pltpu.make_async_copy(dst3d.at[sid], idxd, isem).wait()

        def substep(i, b):
            buf, b2 = bufs[b], (b - S) % B

            @pl.when(i < NCHT)
            def _():
                pltpu.make_async_copy(
                    h_src.at[idxs.at[i]], buf, gsems[b]).wait()
                pltpu.async_copy(buf, acc.at[idxd.at[i]], ssems[b], add=True)

                @pl.when(i + B - S < NCHT)
                def _():
                    @pl.when(i >= S)
                    def _():
                        pltpu.make_async_copy(
                            bufs[b2], acc.at[idxd.at[i - S]],
                            ssems[b2]).wait()

                    pltpu.async_copy(
                        h_src.at[idxs.at[i + B - S]], bufs[b2], gsems[b2])

        for i in range(B - S):
            pltpu.async_copy(h_src.at[idxs.at[i]], bufs[i], gsems[i])

        def step(j, _):
            for b in range(B):
                substep(B * j + b, b)
            return ()

        lax.fori_loop(0, (NCHT + B - 1) // B, step, (), unroll=False)
        for i in range(NCHT - B, NCHT):
            b = i % B
            pltpu.make_async_copy(
                bufs[b], acc.at[idxd.at[i]], ssems[b]).wait()
        plsc.subcore_barrier()

        pltpu.sync_copy(acc.at[pl.ds(rbase, RPS)], out.at[pl.ds(rbase, RPS)])

        @pl.when(sid == NS - 1)
        def _():
            pltpu.sync_copy(acc.at[pl.ds(NS * RPS, RREM)],
                            out.at[pl.ds(NS * RPS, RREM)])

    @pl.when(cid == 0)
    def _():
        run(hs0, hi0, out0)

    @pl.when(cid == 1)
    def _():
        run(hs1, hi1, out1)


_BM = 2000


def _mm_in_body(x_ref, w_ref, o0_ref, o1_ref):
    h = jnp.maximum(
        jnp.dot(x_ref[...], w_ref[...], preferred_element_type=jnp.float32),
        0.0)
    o0_ref[...] = h[:, :KH]
    o1_ref[...] = h[:, KH:]


def _mm_in(x, w):
    m = x.shape[0]
    half = jax.ShapeDtypeStruct((m, KH), jnp.float32)
    ospec = pl.BlockSpec((_BM, KH), lambda i: (i, 0))
    return pl.pallas_call(
        _mm_in_body,
        grid=(m // _BM,),
        in_specs=[
            pl.BlockSpec((_BM, DH), lambda i: (i, 0)),
            pl.BlockSpec((DH, DH), lambda i: (0, 0)),
        ],
        out_specs=(ospec, ospec),
        out_shape=(half, half),
    )(x, w)


def _mm_mid_body(x0_ref, x1_ref, w_ref, o0_ref, o1_ref):
    x = jnp.concatenate([x0_ref[...], x1_ref[...]], axis=1)
    h = jnp.maximum(
        jnp.dot(x, w_ref[...] * 0.5, preferred_element_type=jnp.float32),
        0.0)
    o0_ref[...] = h[:, :KH]
    o1_ref[...] = h[:, KH:]


def _mm_mid(x0, x1, w):
    m = x0.shape[0]
    half = jax.ShapeDtypeStruct((m, KH), jnp.float32)
    xspec = pl.BlockSpec((_BM, KH), lambda i: (i, 0))
    return pl.pallas_call(
        _mm_mid_body,
        grid=(m // _BM,),
        in_specs=[xspec, xspec, pl.BlockSpec((DH, DH), lambda i: (0, 0))],
        out_specs=(xspec, xspec),
        out_shape=(half, half),
    )(x0, x1, w)


def _mm_head_body(x0_ref, x1_ref, w_ref, b_ref, o_ref):
    x = jnp.concatenate([x0_ref[...], x1_ref[...]], axis=1)
    o_ref[...] = jnp.dot(
        x, w_ref[...] * 0.5, preferred_element_type=jnp.float32) + b_ref[...]


def _mm_head(x0, x1, w, b):
    m = x0.shape[0]
    c = w.shape[1]
    xspec = pl.BlockSpec((_BM, KH), lambda i: (i, 0))
    return pl.pallas_call(
        _mm_head_body,
        grid=(m // _BM,),
        in_specs=[
            xspec, xspec,
            pl.BlockSpec((DH, c), lambda i: (0, 0)),
            pl.BlockSpec((1, c), lambda i: (0, 0)),
        ],
        out_specs=pl.BlockSpec((_BM, c), lambda i: (i, 0)),
        out_shape=jax.ShapeDtypeStruct((m, c), jnp.float32),
    )(x0, x1, w, b.reshape(1, c))


def kernel(x_paper, x_author, edge_index_writes, edge_index_rev_writes,
           W_paper_0, W_paper_1, W_author_0, W_author_1, W_out, b_out):
    src_w = edge_index_writes[0].reshape(NS, NCHT, K)
    dst_w = edge_index_writes[1].reshape(NS, NCHT, K)
    src_r = edge_index_rev_writes[0].reshape(NS, NCHT, K)
    dst_r = edge_index_rev_writes[1].reshape(NS, NCHT, K)

    hp0_0, hp0_1 = _mm_in(x_paper, W_paper_0)
    ha0_0, ha0_1 = _mm_in(x_author, W_author_0)
    pp_0, pp_1 = _spmm_kernel(ha0_0, ha0_1, hp0_0, hp0_1, src_w, dst_w)
    pa_0, pa_1 = _spmm_kernel(hp0_0, hp0_1, ha0_0, ha0_1, src_r, dst_r)
    hp1_0, hp1_1 = _mm_mid(pp_0, pp_1, W_paper_1)
    ha1_0, ha1_1 = _mm_mid(pa_0, pa_1, W_author_1)
    qq_0, qq_1 = _spmm_kernel(ha1_0, ha1_1, hp1_0, hp1_1, src_w, dst_w)
    return _mm_head(qq_0, qq_1, W_out, b_out)


# speedup vs baseline: 11.2038x; 1.0398x over previous
import functools

import jax
import jax.numpy as jnp
from jax import lax
from jax.experimental import pallas as pl
from jax.experimental.pallas import tpu as pltpu
from jax.experimental.pallas import tpu_sc as plsc

N = 10000
DH = 128
KH = DH // 2
E = 320000
NC = 2
NS = 16
EPT = E // NS
K = 80
NCHT = EPT // K
RPS = 624
RREM = N - NS * RPS

_mesh = plsc.VectorSubcoreMesh(core_axis_name="c", subcore_axis_name="s")


@functools.partial(
    pl.kernel,
    out_type=(
        jax.ShapeDtypeStruct((N, KH), jnp.float32),
        jax.ShapeDtypeStruct((N, KH), jnp.float32),
    ),
    mesh=_mesh,
    scratch_types=[
        pltpu.VMEM_SHARED((N, KH), jnp.float32),
        pltpu.VMEM((NCHT, K), jnp.int32),
        pltpu.VMEM((NCHT, K), jnp.int32),
    ] + [pltpu.VMEM((K, KH), jnp.float32) for _ in range(8)]
      + [pltpu.SemaphoreType.DMA for _ in range(17)],
    compiler_params=pltpu.CompilerParams(use_tc_tiling_on_sc=False),
)
def _spmm_kernel(hs0, hs1, hi0, hi1, src3d, dst3d, out0, out1,
                 acc, idxs, idxd, *ring):
    cid = lax.axis_index("c")
    sid = lax.axis_index("s")
    rbase = sid * RPS

    isem = ring[-1]
    pltpu.async_copy(src3d.at[sid], idxs, isem)
    pltpu.async_copy(dst3d.at[sid], idxd, isem)

    def run(h_src, h_init, out):
        pltpu.sync_copy(h_init.at[pl.ds(rbase, RPS)],
                        acc.at[pl.ds(rbase, RPS)])

        @pl.when(sid == NS - 1)
        def _():
            pltpu.sync_copy(h_init.at[pl.ds(NS * RPS, RREM)],
                            acc.at[pl.ds(NS * RPS, RREM)])

        plsc.subcore_barrier()

        B, S = 8, 4
        bufs = ring[:B]
        gsems = ring[B:2 * B]
        ssems = ring[2 * B:3 * B]
        pltpu.make_async_copy(src3d.at[sid], idxs, isem).wait()
        pltpu.make_async_copy(dst3d.at[sid], idxd, isem).wait()

        def substep(i, b):
            buf, b2 = bufs[b], (b - S) % B

            @pl.when(i < NCHT)
            def _():
                pltpu.make_async_copy(
                    h_src.at[idxs.at[i]], buf, gsems[b]).wait()
                pltpu.async_copy(buf, acc.at[idxd.at[i]], ssems[b], add=True)

                @pl.when(i + B - S < NCHT)
                def _():
                    @pl.when(i >= S)
                    def _():
                        pltpu.make_async_copy(
                            bufs[b2], acc.at[idxd.at[i - S]],
                            ssems[b2]).wait()

                    pltpu.async_copy(
                        h_src.at[idxs.at[i + B - S]], bufs[b2], gsems[b2])

        for i in range(B - S):
            pltpu.async_copy(h_src.at[idxs.at[i]], bufs[i], gsems[i])

        def step(j, _):
            for b in range(B):
                substep(B * j + b, b)
            return ()

        lax.fori_loop(0, (NCHT + B - 1) // B, step, (), unroll=False)
        for i in range(NCHT - B, NCHT):
            b = i % B
            pltpu.make_async_copy(
                bufs[b], acc.at[idxd.at[i]], ssems[b]).wait()
        plsc.subcore_barrier()

        pltpu.sync_copy(acc.at[pl.ds(rbase, RPS)], out.at[pl.ds(rbase, RPS)])

        @pl.when(sid == NS - 1)
        def _():
            pltpu.sync_copy(acc.at[pl.ds(NS * RPS, RREM)],
                            out.at[pl.ds(NS * RPS, RREM)])

    @pl.when(cid == 0)
    def _():
        run(hs0, hi0, out0)

    @pl.when(cid == 1)
    def _():
        run(hs1, hi1, out1)


_BM = 2000


def _mm_in_body(x_ref, w_ref, o0_ref, o1_ref):
    h = jnp.maximum(
        jnp.dot(x_ref[...], w_ref[...], preferred_element_type=jnp.float32),
        0.0)
    o0_ref[...] = h[:, :KH]
    o1_ref[...] = h[:, KH:]


def _mm_in(x, w):
    m = x.shape[0]
    half = jax.ShapeDtypeStruct((m, KH), jnp.float32)
    ospec = pl.BlockSpec((_BM, KH), lambda i: (i, 0))
    return pl.pallas_call(
        _mm_in_body,
        grid=(m // _BM,),
        in_specs=[
            pl.BlockSpec((_BM, DH), lambda i: (i, 0)),
            pl.BlockSpec((DH, DH), lambda i: (0, 0)),
        ],
        out_specs=(ospec, ospec),
        out_shape=(half, half),
    )(x, w)


def _mm_mid_body(x0_ref, x1_ref, w_ref, o0_ref, o1_ref):
    x = jnp.concatenate([x0_ref[...], x1_ref[...]], axis=1)
    h = jnp.maximum(
        jnp.dot(x, w_ref[...] * 0.5, preferred_element_type=jnp.float32),
        0.0)
    o0_ref[...] = h[:, :KH]
    o1_ref[...] = h[:, KH:]


def _mm_mid(x0, x1, w):
    m = x0.shape[0]
    half = jax.ShapeDtypeStruct((m, KH), jnp.float32)
    xspec = pl.BlockSpec((_BM, KH), lambda i: (i, 0))
    return pl.pallas_call(
        _mm_mid_body,
        grid=(m // _BM,),
        in_specs=[xspec, xspec, pl.BlockSpec((DH, DH), lambda i: (0, 0))],
        out_specs=(xspec, xspec),
        out_shape=(half, half),
    )(x0, x1, w)


def _mm_head_body(q0_ref, q1_ref, h0_ref, h1_ref, w_ref, b_ref, o_ref):
    x = jnp.concatenate([q0_ref[...] + h0_ref[...],
                         q1_ref[...] + h1_ref[...]], axis=1)
    o_ref[...] = jnp.dot(
        x, w_ref[...] * 0.5, preferred_element_type=jnp.float32) + b_ref[...]


def _mm_head(q0, q1, h0, h1, w, b):
    m = q0.shape[0]
    c = w.shape[1]
    xspec = pl.BlockSpec((_BM, KH), lambda i: (i, 0))
    return pl.pallas_call(
        _mm_head_body,
        grid=(m // _BM,),
        in_specs=[
            xspec, xspec, xspec, xspec,
            pl.BlockSpec((DH, c), lambda i: (0, 0)),
            pl.BlockSpec((1, c), lambda i: (0, 0)),
        ],
        out_specs=pl.BlockSpec((_BM, c), lambda i: (i, 0)),
        out_shape=jax.ShapeDtypeStruct((m, c), jnp.float32),
    )(q0, q1, h0, h1, w, b.reshape(1, c))


def kernel(x_paper, x_author, edge_index_writes, edge_index_rev_writes,
           W_paper_0, W_paper_1, W_author_0, W_author_1, W_out, b_out):
    src_w = edge_index_writes[0].reshape(NS, NCHT, K)
    dst_w = edge_index_writes[1].reshape(NS, NCHT, K)
    src_r = edge_index_rev_writes[0].reshape(NS, NCHT, K)
    dst_r = edge_index_rev_writes[1].reshape(NS, NCHT, K)

    hp0_0, hp0_1 = _mm_in(x_paper, W_paper_0)
    ha0_0, ha0_1 = _mm_in(x_author, W_author_0)
    pa_0, pa_1 = _spmm_kernel(hp0_0, hp0_1, ha0_0, ha0_1, src_r, dst_r)
    pp_0, pp_1 = _spmm_kernel(ha0_0, ha0_1, hp0_0, hp0_1, src_w, dst_w)
    ha1_0, ha1_1 = _mm_mid(pa_0, pa_1, W_author_1)
    hp1_0, hp1_1 = _mm_mid(pp_0, pp_1, W_paper_1)
    z = jnp.zeros((N, KH), jnp.float32)
    qq_0, qq_1 = _spmm_kernel(ha1_0, ha1_1, z, z, src_w, dst_w)
    return _mm_head(qq_0, qq_1, hp1_0, hp1_1, W_out, b_out)


# speedup vs baseline: 12.6406x; 1.1282x over previous
import functools

import jax
import jax.numpy as jnp
from jax import lax
from jax.experimental import pallas as pl
from jax.experimental.pallas import tpu as pltpu
from jax.experimental.pallas import tpu_sc as plsc

N = 10000
DH = 128
KH = DH // 2
E = 320000
NC = 2
NS = 16
EPT = E // NS
K = 80
NCHT = EPT // K
RPS = 624
RREM = N - NS * RPS

_mesh = plsc.VectorSubcoreMesh(core_axis_name="c", subcore_axis_name="s")


@functools.partial(
    pl.kernel,
    out_type=jax.ShapeDtypeStruct((N, DH), jnp.float32),
    mesh=_mesh,
    scratch_types=[
        pltpu.VMEM_SHARED((N, KH), jnp.float32),
        pltpu.VMEM((NCHT, K), jnp.int32),
        pltpu.VMEM((NCHT, K), jnp.int32),
    ] + [pltpu.VMEM((K, KH), jnp.float32) for _ in range(8)]
      + [pltpu.SemaphoreType.DMA for _ in range(17)],
    compiler_params=pltpu.CompilerParams(use_tc_tiling_on_sc=False),
)
def _spmm_kernel(h_src_v, src3d, dst3d, out, acc, idxs, idxd, *ring):
    cid = lax.axis_index("c")
    sid = lax.axis_index("s")
    rbase = sid * RPS

    isem = ring[-1]
    pltpu.async_copy(src3d.at[sid], idxs, isem)
    pltpu.async_copy(dst3d.at[sid], idxd, isem)

    hv = h_src_v.at[pl.ds(cid, 2 * N - 1)]

    B, S = 8, 4
    bufs = ring[:B]
    gsems = ring[B:2 * B]
    ssems = ring[2 * B:3 * B]

    z16 = jnp.zeros((16,), jnp.float32)
    for r in range(K):
        for t in range(KH // 16):
            bufs[0][r, pl.ds(16 * t, 16)] = z16
    for r in range(RPS // K):
        pltpu.sync_copy(bufs[0], acc.at[pl.ds(rbase + K * r, K)])
    pltpu.sync_copy(bufs[0].at[pl.ds(0, RPS % K)],
                    acc.at[pl.ds(rbase + K * (RPS // K), RPS % K)])

    @pl.when(sid == NS - 1)
    def _():
        pltpu.sync_copy(bufs[0].at[pl.ds(0, RREM)],
                        acc.at[pl.ds(NS * RPS, RREM)])

    plsc.subcore_barrier()
    pltpu.make_async_copy(src3d.at[sid], idxs, isem).wait()
    pltpu.make_async_copy(dst3d.at[sid], idxd, isem).wait()

    def substep(i, b):
        buf, b2 = bufs[b], (b - S) % B

        @pl.when(i < NCHT)
        def _():
            pltpu.make_async_copy(hv.at[idxs.at[i]], buf, gsems[b]).wait()
            pltpu.async_copy(buf, acc.at[idxd.at[i]], ssems[b], add=True)

            @pl.when(i + B - S < NCHT)
            def _():
                @pl.when(i >= S)
                def _():
                    pltpu.make_async_copy(
                        bufs[b2], acc.at[idxd.at[i - S]], ssems[b2]).wait()

                pltpu.async_copy(
                    hv.at[idxs.at[i + B - S]], bufs[b2], gsems[b2])

    for i in range(B - S):
        pltpu.async_copy(hv.at[idxs.at[i]], bufs[i], gsems[i])

    def step(j, _):
        for b in range(B):
            substep(B * j + b, b)
        return ()

    lax.fori_loop(0, (NCHT + B - 1) // B, step, (), unroll=False)
    for i in range(NCHT - B, NCHT):
        b = i % B
        pltpu.make_async_copy(bufs[b], acc.at[idxd.at[i]], ssems[b]).wait()
    plsc.subcore_barrier()

    pltpu.sync_copy(acc.at[pl.ds(rbase, RPS)],
                    out.at[pl.ds(rbase, RPS), pl.ds(cid * KH, KH)])

    @pl.when(sid == NS - 1)
    def _():
        pltpu.sync_copy(acc.at[pl.ds(NS * RPS, RREM)],
                        out.at[pl.ds(NS * RPS, RREM), pl.ds(cid * KH, KH)])


_BM = 2000


def _mm_body(x_ref, w_ref, o_ref):
    o_ref[...] = jnp.maximum(
        jnp.dot(x_ref[...], w_ref[...], preferred_element_type=jnp.float32),
        0.0)


def _mm_relu(x, w):
    m, d = x.shape
    dout = w.shape[1]
    return pl.pallas_call(
        _mm_body,
        grid=(m // _BM,),
        in_specs=[
            pl.BlockSpec((_BM, d), lambda i: (i, 0)),
            pl.BlockSpec((d, dout), lambda i: (0, 0)),
        ],
        out_specs=pl.BlockSpec((_BM, dout), lambda i: (i, 0)),
        out_shape=jax.ShapeDtypeStruct((m, dout), jnp.float32),
    )(x, w)


def _mm_add_body(q_ref, h_ref, w_ref, o_ref):
    o_ref[...] = jnp.maximum(
        jnp.dot(q_ref[...] + h_ref[...], w_ref[...] * 0.5,
                preferred_element_type=jnp.float32),
        0.0)


def _mm_add_relu(q, h, w):
    m, d = q.shape
    dout = w.shape[1]
    xspec = pl.BlockSpec((_BM, d), lambda i: (i, 0))
    return pl.pallas_call(
        _mm_add_body,
        grid=(m // _BM,),
        in_specs=[xspec, xspec, pl.BlockSpec((d, dout), lambda i: (0, 0))],
        out_specs=pl.BlockSpec((_BM, dout), lambda i: (i, 0)),
        out_shape=jax.ShapeDtypeStruct((m, dout), jnp.float32),
    )(q, h, w)


def _mm_head_body(q_ref, h_ref, w_ref, b_ref, o_ref):
    o_ref[...] = jnp.dot(
        q_ref[...] + h_ref[...], w_ref[...] * 0.5,
        preferred_element_type=jnp.float32) + b_ref[...]


def _mm_head(q, h, w, b):
    m, d = q.shape
    c = w.shape[1]
    xspec = pl.BlockSpec((_BM, d), lambda i: (i, 0))
    return pl.pallas_call(
        _mm_head_body,
        grid=(m // _BM,),
        in_specs=[
            xspec, xspec,
            pl.BlockSpec((d, c), lambda i: (0, 0)),
            pl.BlockSpec((1, c), lambda i: (0, 0)),
        ],
        out_specs=pl.BlockSpec((_BM, c), lambda i: (i, 0)),
        out_shape=jax.ShapeDtypeStruct((m, c), jnp.float32),
    )(q, h, w, b.reshape(1, c))


def kernel(x_paper, x_author, edge_index_writes, edge_index_rev_writes,
           W_paper_0, W_paper_1, W_author_0, W_author_1, W_out, b_out):
    src_w = (edge_index_writes[0] * 2).reshape(NS, NCHT, K)
    dst_w = edge_index_writes[1].reshape(NS, NCHT, K)
    src_r = (edge_index_rev_writes[0] * 2).reshape(NS, NCHT, K)
    dst_r = edge_index_rev_writes[1].reshape(NS, NCHT, K)

    hp0 = _mm_relu(x_paper, W_paper_0)
    ha0 = _mm_relu(x_author, W_author_0)
    pa = _spmm_kernel(hp0.reshape(2 * N, KH), src_r, dst_r)
    pp = _spmm_kernel(ha0.reshape(2 * N, KH), src_w, dst_w)
    ha1 = _mm_add_relu(pa, ha0, W_author_1)
    hp1 = _mm_add_relu(pp, hp0, W_paper_1)
    qq = _spmm_kernel(ha1.reshape(2 * N, KH), src_w, dst_w)
    return _mm_head(qq, hp1, W_out, b_out)


# speedup vs baseline: 12.6592x; 1.0015x over previous
import functools

import jax
import jax.numpy as jnp
from jax import lax
from jax.experimental import pallas as pl
from jax.experimental.pallas import tpu as pltpu
from jax.experimental.pallas import tpu_sc as plsc

N = 10000
DH = 128
KH = DH // 2
E = 320000
NC = 2
NS = 16
EPT = E // NS
K = 80
NCHT = EPT // K
RPS = 624
RREM = N - NS * RPS

_mesh = plsc.VectorSubcoreMesh(core_axis_name="c", subcore_axis_name="s")


@functools.partial(
    pl.kernel,
    out_type=jax.ShapeDtypeStruct((N, DH), jnp.float32),
    mesh=_mesh,
    scratch_types=[
        pltpu.VMEM_SHARED((N, KH), jnp.float32),
        pltpu.VMEM((NCHT, K), jnp.int32),
        pltpu.VMEM((NCHT, K), jnp.int32),
    ] + [pltpu.VMEM((K, KH), jnp.float32) for _ in range(8)]
      + [pltpu.SemaphoreType.DMA for _ in range(17)],
    compiler_params=pltpu.CompilerParams(use_tc_tiling_on_sc=False),
)
def _spmm_kernel(h_src_v, src3d, dst3d, out, acc, idxs, idxd, *ring):
    cid = lax.axis_index("c")
    sid = lax.axis_index("s")
    rbase = sid * RPS

    isem = ring[-1]
    pltpu.async_copy(src3d.at[sid], idxs, isem)
    pltpu.async_copy(dst3d.at[sid], idxd, isem)

    hv = h_src_v.at[pl.ds(cid, 2 * N - 1)]

    B, S = 8, 4
    bufs = ring[:B]
    gsems = ring[B:2 * B]
    ssems = ring[2 * B:3 * B]

    z16 = jnp.zeros((16,), jnp.float32)
    for r in range(K):
        for t in range(KH // 16):
            bufs[0][r, pl.ds(16 * t, 16)] = z16
    zcopies = []
    for r in range(RPS // K):
        zcopies.append(pltpu.make_async_copy(
            bufs[0], acc.at[pl.ds(rbase + K * r, K)], ssems[r]))
    zcopies.append(pltpu.make_async_copy(
        bufs[0].at[pl.ds(0, RPS % K)],
        acc.at[pl.ds(rbase + K * (RPS // K), RPS % K)],
        ssems[RPS // K]))
    for c in zcopies:
        c.start()

    @pl.when(sid == NS - 1)
    def _():
        pltpu.sync_copy(bufs[0].at[pl.ds(0, RREM)],
                        acc.at[pl.ds(NS * RPS, RREM)])

    for c in zcopies:
        c.wait()
    plsc.subcore_barrier()
    pltpu.make_async_copy(src3d.at[sid], idxs, isem).wait()
    pltpu.make_async_copy(dst3d.at[sid], idxd, isem).wait()

    def substep(i, b):
        buf, b2 = bufs[b], (b - S) % B

        @pl.when(i < NCHT)
        def _():
            pltpu.make_async_copy(hv.at[idxs.at[i]], buf, gsems[b]).wait()
            pltpu.async_copy(buf, acc.at[idxd.at[i]], ssems[b], add=True)

            @pl.when(i + B - S < NCHT)
            def _():
                @pl.when(i >= S)
                def _():
                    pltpu.make_async_copy(
                        bufs[b2], acc.at[idxd.at[i - S]], ssems[b2]).wait()

                pltpu.async_copy(
                    hv.at[idxs.at[i + B - S]], bufs[b2], gsems[b2])

    for i in range(B - S):
        pltpu.async_copy(hv.at[idxs.at[i]], bufs[i], gsems[i])

    def step(j, _):
        for b in range(B):
            substep(B * j + b, b)
        return ()

    lax.fori_loop(0, (NCHT + B - 1) // B, step, (), unroll=False)
    for i in range(NCHT - B, NCHT):
        b = i % B
        pltpu.make_async_copy(bufs[b], acc.at[idxd.at[i]], ssems[b]).wait()
    plsc.subcore_barrier()

    pltpu.sync_copy(acc.at[pl.ds(rbase, RPS)],
                    out.at[pl.ds(rbase, RPS), pl.ds(cid * KH, KH)])

    @pl.when(sid == NS - 1)
    def _():
        pltpu.sync_copy(acc.at[pl.ds(NS * RPS, RREM)],
                        out.at[pl.ds(NS * RPS, RREM), pl.ds(cid * KH, KH)])


_BM = 2000


def _mm_body(x_ref, w_ref, o_ref):
    o_ref[...] = jnp.maximum(
        jnp.dot(x_ref[...], w_ref[...], preferred_element_type=jnp.float32),
        0.0)


def _mm_relu(x, w):
    m, d = x.shape
    dout = w.shape[1]
    return pl.pallas_call(
        _mm_body,
        grid=(m // _BM,),
        in_specs=[
            pl.BlockSpec((_BM, d), lambda i: (i, 0)),
            pl.BlockSpec((d, dout), lambda i: (0, 0)),
        ],
        out_specs=pl.BlockSpec((_BM, dout), lambda i: (i, 0)),
        out_shape=jax.ShapeDtypeStruct((m, dout), jnp.float32),
    )(x, w)


def _mm_add_body(q_ref, h_ref, w_ref, o_ref):
    o_ref[...] = jnp.maximum(
        jnp.dot(q_ref[...] + h_ref[...], w_ref[...] * 0.5,
                preferred_element_type=jnp.float32),
        0.0)


def _mm_add_relu(q, h, w):
    m, d = q.shape
    dout = w.shape[1]
    xspec = pl.BlockSpec((_BM, d), lambda i: (i, 0))
    return pl.pallas_call(
        _mm_add_body,
        grid=(m // _BM,),
        in_specs=[xspec, xspec, pl.BlockSpec((d, dout), lambda i: (0, 0))],
        out_specs=pl.BlockSpec((_BM, dout), lambda i: (i, 0)),
        out_shape=jax.ShapeDtypeStruct((m, dout), jnp.float32),
    )(q, h, w)


def _mm_head_body(q_ref, h_ref, w_ref, b_ref, o_ref):
    o_ref[...] = jnp.dot(
        q_ref[...] + h_ref[...], w_ref[...] * 0.5,
        preferred_element_type=jnp.float32) + b_ref[...]


def _mm_head(q, h, w, b):
    m, d = q.shape
    c = w.shape[1]
    xspec = pl.BlockSpec((_BM, d), lambda i: (i, 0))
    return pl.pallas_call(
        _mm_head_body,
        grid=(m // _BM,),
        in_specs=[
            xspec, xspec,
            pl.BlockSpec((d, c), lambda i: (0, 0)),
            pl.BlockSpec((1, c), lambda i: (0, 0)),
        ],
        out_specs=pl.BlockSpec((_BM, c), lambda i: (i, 0)),
        out_shape=jax.ShapeDtypeStruct((m, c), jnp.float32),
    )(q, h, w, b.reshape(1, c))


def kernel(x_paper, x_author, edge_index_writes, edge_index_rev_writes,
           W_paper_0, W_paper_1, W_author_0, W_author_1, W_out, b_out):
    src_w = (edge_index_writes[0] * 2).reshape(NS, NCHT, K)
    dst_w = edge_index_writes[1].reshape(NS, NCHT, K)
    src_r = (edge_index_rev_writes[0] * 2).reshape(NS, NCHT, K)
    dst_r = edge_index_rev_writes[1].reshape(NS, NCHT, K)

    hp0 = _mm_relu(x_paper, W_paper_0)
    ha0 = _mm_relu(x_author, W_author_0)
    pa = _spmm_kernel(hp0.reshape(2 * N, KH), src_r, dst_r)
    pp = _spmm_kernel(ha0.reshape(2 * N, KH), src_w, dst_w)
    ha1 = _mm_add_relu(pa, ha0, W_author_1)
    hp1 = _mm_add_relu(pp, hp0, W_paper_1)
    qq = _spmm_kernel(ha1.reshape(2 * N, KH), src_w, dst_w)
    return _mm_head(qq, hp1, W_out, b_out)


# speedup vs baseline: 13.0989x; 1.0347x over previous
import functools

import jax
import jax.numpy as jnp
from jax import lax
from jax.experimental import pallas as pl
from jax.experimental.pallas import tpu as pltpu
from jax.experimental.pallas import tpu_sc as plsc

N = 10000
DH = 128
KH = DH // 2
E = 320000
NC = 2
NS = 16
EPT = E // NS
K = 80
NCHT = EPT // K
RPS = 624
RREM = N - NS * RPS

_mesh = plsc.VectorSubcoreMesh(core_axis_name="c", subcore_axis_name="s")


@functools.partial(
    pl.kernel,
    out_type=jax.ShapeDtypeStruct((N, DH), jnp.float32),
    mesh=_mesh,
    scratch_types=[
        pltpu.VMEM_SHARED((N, KH), jnp.float32),
        pltpu.VMEM((NCHT, K), jnp.int32),
        pltpu.VMEM((NCHT, K), jnp.int32),
    ] + [pltpu.VMEM((K, KH), jnp.float32) for _ in range(9)]
      + [pltpu.SemaphoreType.DMA for _ in range(19)],
    compiler_params=pltpu.CompilerParams(use_tc_tiling_on_sc=False),
)
def _spmm_kernel(h_src_v, src3d, dst3d, out, acc, idxs, idxd, *ring):
    cid = lax.axis_index("c")
    sid = lax.axis_index("s")
    rbase = sid * RPS

    isem = ring[-1]
    pltpu.async_copy(src3d.at[sid], idxs, isem)
    pltpu.async_copy(dst3d.at[sid], idxd, isem)

    hv = h_src_v.at[pl.ds(cid, 2 * N - 1)]

    B, S = 9, 4
    bufs = ring[:B]
    gsems = ring[B:2 * B]
    ssems = ring[2 * B:3 * B]

    z16 = jnp.zeros((16,), jnp.float32)
    for r in range(K):
        for t in range(KH // 16):
            bufs[0][r, pl.ds(16 * t, 16)] = z16
    zcopies = []
    for r in range(RPS // K):
        zcopies.append(pltpu.make_async_copy(
            bufs[0], acc.at[pl.ds(rbase + K * r, K)], ssems[r]))
    zcopies.append(pltpu.make_async_copy(
        bufs[0].at[pl.ds(0, RPS % K)],
        acc.at[pl.ds(rbase + K * (RPS // K), RPS % K)],
        ssems[RPS // K]))
    for c in zcopies:
        c.start()

    @pl.when(sid == NS - 1)
    def _():
        pltpu.sync_copy(bufs[0].at[pl.ds(0, RREM)],
                        acc.at[pl.ds(NS * RPS, RREM)])

    for c in zcopies:
        c.wait()
    plsc.subcore_barrier()
    pltpu.make_async_copy(src3d.at[sid], idxs, isem).wait()
    pltpu.make_async_copy(dst3d.at[sid], idxd, isem).wait()

    def substep(i, b):
        buf, b2 = bufs[b], (b - S) % B

        @pl.when(i < NCHT)
        def _():
            pltpu.make_async_copy(hv.at[idxs.at[i]], buf, gsems[b]).wait()
            pltpu.async_copy(buf, acc.at[idxd.at[i]], ssems[b], add=True)

            @pl.when(i + B - S < NCHT)
            def _():
                @pl.when(i >= S)
                def _():
                    pltpu.make_async_copy(
                        bufs[b2], acc.at[idxd.at[i - S]], ssems[b2]).wait()

                pltpu.async_copy(
                    hv.at[idxs.at[i + B - S]], bufs[b2], gsems[b2])

    for i in range(B - S):
        pltpu.async_copy(hv.at[idxs.at[i]], bufs[i], gsems[i])

    def step(j, _):
        for b in range(B):
            substep(B * j + b, b)
        return ()

    lax.fori_loop(0, (NCHT + B - 1) // B, step, (), unroll=False)
    for i in range(NCHT - B, NCHT):
        b = i % B
        pltpu.make_async_copy(bufs[b], acc.at[idxd.at[i]], ssems[b]).wait()
    plsc.subcore_barrier()

    pltpu.sync_copy(acc.at[pl.ds(rbase, RPS)],
                    out.at[pl.ds(rbase, RPS), pl.ds(cid * KH, KH)])

    @pl.when(sid == NS - 1)
    def _():
        pltpu.sync_copy(acc.at[pl.ds(NS * RPS, RREM)],
                        out.at[pl.ds(NS * RPS, RREM), pl.ds(cid * KH, KH)])


_BM = 2000


def _mm_body(x_ref, w_ref, o_ref):
    o_ref[...] = jnp.maximum(
        jnp.dot(x_ref[...], w_ref[...], preferred_element_type=jnp.float32),
        0.0)


def _mm_relu(x, w):
    m, d = x.shape
    dout = w.shape[1]
    return pl.pallas_call(
        _mm_body,
        grid=(m // _BM,),
        in_specs=[
            pl.BlockSpec((_BM, d), lambda i: (i, 0)),
            pl.BlockSpec((d, dout), lambda i: (0, 0)),
        ],
        out_specs=pl.BlockSpec((_BM, dout), lambda i: (i, 0)),
        out_shape=jax.ShapeDtypeStruct((m, dout), jnp.float32),
    )(x, w)


def _mm_add_body(q_ref, h_ref, w_ref, o_ref):
    o_ref[...] = jnp.maximum(
        jnp.dot(q_ref[...] + h_ref[...], w_ref[...] * 0.5,
                preferred_element_type=jnp.float32),
        0.0)


def _mm_add_relu(q, h, w):
    m, d = q.shape
    dout = w.shape[1]
    xspec = pl.BlockSpec((_BM, d), lambda i: (i, 0))
    return pl.pallas_call(
        _mm_add_body,
        grid=(m // _BM,),
        in_specs=[xspec, xspec, pl.BlockSpec((d, dout), lambda i: (0, 0))],
        out_specs=pl.BlockSpec((_BM, dout), lambda i: (i, 0)),
        out_shape=jax.ShapeDtypeStruct((m, dout), jnp.float32),
    )(q, h, w)


def _mm_head_body(q_ref, h_ref, w_ref, b_ref, o_ref):
    o_ref[...] = jnp.dot(
        q_ref[...] + h_ref[...], w_ref[...] * 0.5,
        preferred_element_type=jnp.float32) + b_ref[...]


def _mm_head(q, h, w, b):
    m, d = q.shape
    c = w.shape[1]
    xspec = pl.BlockSpec((_BM, d), lambda i: (i, 0))
    return pl.pallas_call(
        _mm_head_body,
        grid=(m // _BM,),
        in_specs=[
            xspec, xspec,
            pl.BlockSpec((d, c), lambda i: (0, 0)),
            pl.BlockSpec((1, c), lambda i: (0, 0)),
        ],
        out_specs=pl.BlockSpec((_BM, c), lambda i: (i, 0)),
        out_shape=jax.ShapeDtypeStruct((m, c), jnp.float32),
    )(q, h, w, b.reshape(1, c))


def kernel(x_paper, x_author, edge_index_writes, edge_index_rev_writes,
           W_paper_0, W_paper_1, W_author_0, W_author_1, W_out, b_out):
    src_w = (edge_index_writes[0] * 2).reshape(NS, NCHT, K)
    dst_w = edge_index_writes[1].reshape(NS, NCHT, K)
    src_r = (edge_index_rev_writes[0] * 2).reshape(NS, NCHT, K)
    dst_r = edge_index_rev_writes[1].reshape(NS, NCHT, K)

    hp0 = _mm_relu(x_paper, W_paper_0)
    ha0 = _mm_relu(x_author, W_author_0)
    pa = _spmm_kernel(hp0.reshape(2 * N, KH), src_r, dst_r)
    pp = _spmm_kernel(ha0.reshape(2 * N, KH), src_w, dst_w)
    ha1 = _mm_add_relu(pa, ha0, W_author_1)
    hp1 = _mm_add_relu(pp, hp0, W_paper_1)
    qq = _spmm_kernel(ha1.reshape(2 * N, KH), src_w, dst_w)
    return _mm_head(qq, hp1, W_out, b_out)
